# baseline (speedup 1.0000x reference)
import functools

import jax
import jax.numpy as jnp
from jax import lax
from jax.experimental import pallas as pl
from jax.experimental.pallas import tpu as pltpu
from jax.experimental.pallas import tpu_sc as plsc

N = 10000
E = 320000
D = 128
W = 144
NC, NS, L = 2, 16, 16
NW = NC * NS
EPT = E // NW
CH = 40
IB = 2
BLKE = CH * IB
NBLK = EPT // BLKE
NP = 10240
RPT = NP // NS
NSEG = D // L
MROWS = 1000


def _sc_edge_phase(xl, xr, edge_index, att):
    mesh = plsc.VectorSubcoreMesh(
        core_axis_name="c", subcore_axis_name="s", num_cores=NC, num_subcores=NS)

    @functools.partial(
        pl.kernel,
        out_type=jax.ShapeDtypeStruct((NC, N, W), jnp.float32),
        mesh=mesh,
        scratch_types=[
            pltpu.VMEM_SHARED((NP, W), jnp.float32),
            pltpu.VMEM((3, BLKE), jnp.int32),
            pltpu.VMEM((3, IB, CH), jnp.int32),
            pltpu.VMEM((2, CH, D), jnp.float32),
            pltpu.VMEM((2, CH, D), jnp.float32),
            pltpu.VMEM((2, CH, W), jnp.float32),
            pltpu.VMEM((D,), jnp.float32),
            pltpu.SemaphoreType.DMA,
            pltpu.SemaphoreType.DMA,
            pltpu.SemaphoreType.DMA,
            pltpu.SemaphoreType.DMA,
            pltpu.SemaphoreType.DMA,
            pltpu.SemaphoreType.DMA,
            pltpu.SemaphoreType.DMA,
        ],
        compiler_params=pltpu.CompilerParams(use_tc_tiling_on_sc=False),
    )
    def k(xl_hbm, xr_hbm, ei_hbm, att_hbm, out_hbm,
          acc, src_blk, dst_blk, xl_rows, xr_rows, msg, att_v,
          sem_i, sem_l0, sem_l1, sem_r0, sem_r1, sem_s0, sem_s1):
        cid = lax.axis_index("c")
        sid = lax.axis_index("s")
        wid = sid * NC + cid
        sem_l = [sem_l0, sem_l1]
        sem_r = [sem_r0, sem_r1]
        sem_s = [sem_s0, sem_s1]

        zero16 = jnp.zeros((L,), jnp.float32)
        lanes = lax.iota(jnp.int32, L)
        lane0 = lanes == 0
        perms = [lanes ^ s for s in (8, 4, 2, 1)]

        def zrow(r, carry):
            for kk in range(W // L):
                msg[0, r, pl.ds(kk * L, L)] = zero16
            return carry
        lax.fori_loop(0, CH, zrow, 0)
        r0 = sid * RPT
        zcopies = [pltpu.make_async_copy(msg.at[0],
                                         acc.at[pl.ds(r0 + z * CH, CH)],
                                         sem_i)
                   for z in range(RPT // CH)]
        for cpy in zcopies:
            cpy.start()

        ebase0 = wid * EPT
        pltpu.sync_copy(att_hbm.at[0], att_v)
        att_segs = [att_v[pl.ds(kk * L, L)] for kk in range(NSEG)]
        for cpy in zcopies:
            cpy.wait()

        plsc.subcore_barrier()

        def idx_copies(blk, slot):
            base = ebase0 + blk * BLKE
            cps = [pltpu.make_async_copy(ei_hbm.at[0, pl.ds(base, BLKE)],
                                         src_blk.at[slot], sem_i)]
            for c in range(IB):
                cps.append(pltpu.make_async_copy(
                    ei_hbm.at[1, pl.ds(base + c * CH, CH)],
                    dst_blk.at[slot, c], sem_i))
            return cps

        SPLITS = ((0, 24), (24, 16))

        def gather_starts(c, q, slot):
            for off, sz in SPLITS:
                il = src_blk.at[slot, pl.ds(c * CH + off, sz)]
                ir = dst_blk.at[slot, c, pl.ds(off, sz)]
                pltpu.make_async_copy(xl_hbm.at[il],
                                      xl_rows.at[q, pl.ds(off, sz)],
                                      sem_l[q]).start()
                pltpu.make_async_copy(xr_hbm.at[ir],
                                      xr_rows.at[q, pl.ds(off, sz)],
                                      sem_r[q]).start()

        def gather_copies(c, q, slot):
            il = src_blk.at[slot, pl.ds(c * CH, CH)]
            ir = dst_blk.at[slot, c]
            return (pltpu.make_async_copy(xl_hbm.at[il], xl_rows.at[q],
                                          sem_l[q]),
                    pltpu.make_async_copy(xr_hbm.at[ir], xr_rows.at[q],
                                          sem_r[q]))

        def scat_copy(c, q, slot):
            return pltpu.make_async_copy(msg.at[q],
                                         acc.at[dst_blk.at[slot, c]],
                                         sem_s[q])

        def compute_chunk(q):
            @plsc.parallel_loop(0, CH, 1, unroll=2)
            def bodye(e):
                accv = zero16
                asegs = []
                for kk in range(NSEG):
                    a = xl_rows[q, e, pl.ds(kk * L, L)]
                    b = xr_rows[q, e, pl.ds(kk * L, L)]
                    v = a + b
                    lr = jnp.maximum(v, 0.2 * v)
                    accv = accv + att_segs[kk] * lr
                    asegs.append(a)
                for p in perms:
                    accv = accv + accv.at[p].get(mode="promise_in_bounds")
                exv = jnp.exp(accv)
                for kk in range(NSEG):
                    msg[q, e, pl.ds(kk * L, L)] = exv * asegs[kk]
                msg[q, e, pl.ds(D, L)] = jnp.where(lane0, exv, 0.0)

        for cpy in idx_copies(0, 0):
            cpy.start()
            cpy.wait()
        for c in range(IB):
            gather_starts(c, c, 0)

        def block(b, carry):
            slot = lax.rem(b, 3)
            slotn = lax.rem(b + 1, 3)
            slotp = lax.rem(b + 2, 3)

            @pl.when(b < NBLK - 1)
            def _():
                for cpy in idx_copies(b + 1, slotn):
                    cpy.start()

            for c in range(IB):
                @pl.when(b > 0)
                def _():
                    scat_copy(c, c, slotp).wait()
                gl, gr = gather_copies(c, c, slot)
                gl.wait()
                gr.wait()
                compute_chunk(c)
                scat_copy(c, c, slot).start(add=True)

                @pl.when(b < NBLK - 1)
                def _():
                    if c == 0:
                        for cpy in idx_copies(b + 1, slotn):
                            cpy.wait()
                    gather_starts(c, c, slotn)
            return carry
        lax.fori_loop(0, NBLK, block, 0)

        lastp = (NBLK - 1) % 3
        for c in range(IB):
            scat_copy(c, c, lastp).wait()

        plsc.subcore_barrier()
        @pl.when(sid < NS - 1)
        def _():
            pltpu.sync_copy(acc.at[pl.ds(r0, RPT)],
                            out_hbm.at[cid, pl.ds(r0, RPT)])

        @pl.when(sid == NS - 1)
        def _():
            pltpu.sync_copy(acc.at[pl.ds(r0, N - (NS - 1) * RPT)],
                            out_hbm.at[cid, pl.ds(r0, N - (NS - 1) * RPT)])

    return k(xl, xr, edge_index, att)


def _tc_dual_matmul(xin, Wl, Wr):
    def body(x_ref, wl_ref, wr_ref, ol_ref, or_ref):
        xb = x_ref[...]
        ol_ref[...] = jnp.dot(xb, wl_ref[...],
                              preferred_element_type=jnp.float32)
        or_ref[...] = jnp.dot(xb, wr_ref[...],
                              preferred_element_type=jnp.float32)
    return pl.pallas_call(
        body,
        grid=(N // MROWS,),
        in_specs=[pl.BlockSpec((MROWS, D), lambda i: (i, 0)),
                  pl.BlockSpec((D, D), lambda i: (0, 0)),
                  pl.BlockSpec((D, D), lambda i: (0, 0))],
        out_specs=[pl.BlockSpec((MROWS, D), lambda i: (i, 0)),
                   pl.BlockSpec((MROWS, D), lambda i: (i, 0))],
        out_shape=[jax.ShapeDtypeStruct((N, D), jnp.float32),
                   jax.ShapeDtypeStruct((N, D), jnp.float32)],
    )(xin, Wl, Wr)


def _tc_merge_matmul(acc, b, Wl, Wr):
    def body(a_ref, b_ref, wl_ref, wr_ref, ol_ref, or_ref):
        a = a_ref[0] + a_ref[1]
        den = a[:, D:D + 1] + 1e-16
        h = a[:, :D] / den + b_ref[...]
        h = jnp.maximum(h, 0.1 * h)
        ol_ref[...] = jnp.dot(h, wl_ref[...],
                              preferred_element_type=jnp.float32)
        or_ref[...] = jnp.dot(h, wr_ref[...],
                              preferred_element_type=jnp.float32)
    return pl.pallas_call(
        body,
        grid=(N // MROWS,),
        in_specs=[pl.BlockSpec((NC, MROWS, W), lambda i: (0, i, 0)),
                  pl.BlockSpec((1, D), lambda i: (0, 0)),
                  pl.BlockSpec((D, D), lambda i: (0, 0)),
                  pl.BlockSpec((D, D), lambda i: (0, 0))],
        out_specs=[pl.BlockSpec((MROWS, D), lambda i: (i, 0)),
                   pl.BlockSpec((MROWS, D), lambda i: (i, 0))],
        out_shape=[jax.ShapeDtypeStruct((N, D), jnp.float32),
                   jax.ShapeDtypeStruct((N, D), jnp.float32)],
    )(acc, b, Wl, Wr)


def _tc_final(acc, b):
    def body(a_ref, b_ref, o_ref):
        a = a_ref[0] + a_ref[1]
        den = a[:, D:D + 1] + 1e-16
        z = a[:, :D] / den + b_ref[...]
        o_ref[...] = 1.0 / (1.0 + jnp.exp(-z))
    return pl.pallas_call(
        body,
        grid=(N // MROWS,),
        in_specs=[pl.BlockSpec((NC, MROWS, W), lambda i: (0, i, 0)),
                  pl.BlockSpec((1, D), lambda i: (0, 0))],
        out_specs=pl.BlockSpec((MROWS, D), lambda i: (i, 0)),
        out_shape=jax.ShapeDtypeStruct((N, D), jnp.float32),
    )(acc, b)


def kernel(x, edge_index, Wl1, Wr1, att1, b1, Wl2, Wr2, att2, b2):
    xl1, xr1 = _tc_dual_matmul(x, Wl1, Wr1)
    acc1 = _sc_edge_phase(xl1, xr1, edge_index, att1)
    xl2, xr2 = _tc_merge_matmul(acc1, b1.reshape(1, D), Wl2, Wr2)
    acc2 = _sc_edge_phase(xl2, xr2, edge_index, att2)
    return _tc_final(acc2, b2.reshape(1, D))


# speedup vs baseline: 1.0385x; 1.0385x over previous
import functools

import jax
import jax.numpy as jnp
from jax import lax
from jax.experimental import pallas as pl
from jax.experimental.pallas import tpu as pltpu
from jax.experimental.pallas import tpu_sc as plsc

N = 10000
E = 320000
D = 128
W = 144
NC, NS, L = 2, 16, 16
NW = NC * NS
EPT = E // NW
CH = 40
IB = 2
BLKE = CH * IB
NBLK = EPT // BLKE
NP = 10240
RPT = NP // NS
NSEG = D // L
MROWS = 1000


def _sc_edge_phase(xl, xr, edge_index, att):
    mesh = plsc.VectorSubcoreMesh(
        core_axis_name="c", subcore_axis_name="s", num_cores=NC, num_subcores=NS)

    @functools.partial(
        pl.kernel,
        out_type=[jax.ShapeDtypeStruct((NC, N, D), jnp.float32),
                  jax.ShapeDtypeStruct((NC, N, 8), jnp.float32)],
        mesh=mesh,
        scratch_types=[
            pltpu.VMEM_SHARED((NP, W), jnp.float32),
            pltpu.VMEM((3, BLKE), jnp.int32),
            pltpu.VMEM((3, IB, CH), jnp.int32),
            pltpu.VMEM((2, CH, D), jnp.float32),
            pltpu.VMEM((2, CH, D), jnp.float32),
            pltpu.VMEM((2, CH, W), jnp.float32),
            pltpu.VMEM((D,), jnp.float32),
            pltpu.SemaphoreType.DMA,
            pltpu.SemaphoreType.DMA,
            pltpu.SemaphoreType.DMA,
            pltpu.SemaphoreType.DMA,
            pltpu.SemaphoreType.DMA,
            pltpu.SemaphoreType.DMA,
            pltpu.SemaphoreType.DMA,
        ],
        compiler_params=pltpu.CompilerParams(use_tc_tiling_on_sc=False),
    )
    def k(xl_hbm, xr_hbm, ei_hbm, att_hbm, outm_hbm, outd_hbm,
          acc, src_blk, dst_blk, xl_rows, xr_rows, msg, att_v,
          sem_i, sem_l0, sem_l1, sem_r0, sem_r1, sem_s0, sem_s1):
        cid = lax.axis_index("c")
        sid = lax.axis_index("s")
        wid = sid * NC + cid
        sem_l = [sem_l0, sem_l1]
        sem_r = [sem_r0, sem_r1]
        sem_s = [sem_s0, sem_s1]

        ebase0 = wid * EPT
        zero16 = jnp.zeros((L,), jnp.float32)
        lanes = lax.iota(jnp.int32, L)
        lane0 = lanes == 0
        perms = [lanes ^ s for s in (8, 4, 2, 1)]

        def zrow(r, carry):
            for kk in range(W // L):
                msg[0, r, pl.ds(kk * L, L)] = zero16
            return carry
        lax.fori_loop(0, CH, zrow, 0)
        r0 = sid * RPT
        zcopies = [pltpu.make_async_copy(msg.at[0],
                                         acc.at[pl.ds(r0 + z * CH, CH)],
                                         sem_i)
                   for z in range(RPT // CH)]
        for cpy in zcopies:
            cpy.start()

        pltpu.sync_copy(att_hbm.at[0], att_v)
        att_segs = [att_v[pl.ds(kk * L, L)] for kk in range(NSEG)]
        for cpy in zcopies:
            cpy.wait()

        plsc.subcore_barrier()

        def idx_copies(blk, slot):
            base = ebase0 + blk * BLKE
            cps = [pltpu.make_async_copy(ei_hbm.at[0, pl.ds(base, BLKE)],
                                         src_blk.at[slot], sem_i)]
            for c in range(IB):
                cps.append(pltpu.make_async_copy(
                    ei_hbm.at[1, pl.ds(base + c * CH, CH)],
                    dst_blk.at[slot, c], sem_i))
            return cps

        def gather_copies(c, q, slot):
            il = src_blk.at[slot, pl.ds(c * CH, CH)]
            ir = dst_blk.at[slot, c]
            return (pltpu.make_async_copy(xl_hbm.at[il], xl_rows.at[q],
                                          sem_l[q]),
                    pltpu.make_async_copy(xr_hbm.at[ir], xr_rows.at[q],
                                          sem_r[q]))

        def scat_copy(c, q, slot):
            return pltpu.make_async_copy(msg.at[q],
                                         acc.at[dst_blk.at[slot, c]],
                                         sem_s[q])

        def compute_chunk(q):
            @plsc.parallel_loop(0, CH, 1, unroll=2)
            def bodye(e):
                accv = zero16
                asegs = []
                for kk in range(NSEG):
                    a = xl_rows[q, e, pl.ds(kk * L, L)]
                    b = xr_rows[q, e, pl.ds(kk * L, L)]
                    v = a + b
                    lr = jnp.maximum(v, 0.2 * v)
                    accv = accv + att_segs[kk] * lr
                    asegs.append(a)
                for p in perms:
                    accv = accv + accv.at[p].get(mode="promise_in_bounds")
                exv = jnp.exp(accv)
                for kk in range(NSEG):
                    msg[q, e, pl.ds(kk * L, L)] = exv * asegs[kk]
                msg[q, e, pl.ds(D, L)] = jnp.where(lane0, exv, 0.0)

        for cpy in idx_copies(0, 0):
            cpy.start()
            cpy.wait()
        for c in range(IB):
            for cpy in gather_copies(c, c, 0):
                cpy.start()

        def block(b, carry):
            slot = lax.rem(b, 3)
            slotn = lax.rem(b + 1, 3)
            slotp = lax.rem(b + 2, 3)

            @pl.when(b < NBLK - 1)
            def _():
                for cpy in idx_copies(b + 1, slotn):
                    cpy.start()

            for c in range(IB):
                @pl.when(b > 0)
                def _():
                    scat_copy(c, c, slotp).wait()
                gl, gr = gather_copies(c, c, slot)
                gl.wait()
                gr.wait()
                compute_chunk(c)
                scat_copy(c, c, slot).start(add=True)

                @pl.when(b < NBLK - 1)
                def _():
                    if c == 0:
                        for cpy in idx_copies(b + 1, slotn):
                            cpy.wait()
                    for cpy in gather_copies(c, c, slotn):
                        cpy.start()
            return carry
        lax.fori_loop(0, NBLK, block, 0)

        lastp = (NBLK - 1) % 3
        for c in range(IB):
            scat_copy(c, c, lastp).wait()

        plsc.subcore_barrier()
        @pl.when(sid < NS - 1)
        def _():
            pltpu.sync_copy(acc.at[pl.ds(r0, RPT), pl.ds(0, D)],
                            outm_hbm.at[cid, pl.ds(r0, RPT)])
            pltpu.sync_copy(acc.at[pl.ds(r0, RPT), pl.ds(D, 8)],
                            outd_hbm.at[cid, pl.ds(r0, RPT)])

        @pl.when(sid == NS - 1)
        def _():
            nr = N - (NS - 1) * RPT
            pltpu.sync_copy(acc.at[pl.ds(r0, nr), pl.ds(0, D)],
                            outm_hbm.at[cid, pl.ds(r0, nr)])
            pltpu.sync_copy(acc.at[pl.ds(r0, nr), pl.ds(D, 8)],
                            outd_hbm.at[cid, pl.ds(r0, nr)])

    return k(xl, xr, edge_index, att)


def _tc_dual_matmul(xin, Wl, Wr):
    def body(x_ref, wl_ref, wr_ref, ol_ref, or_ref):
        xb = x_ref[...]
        ol_ref[...] = jnp.dot(xb, wl_ref[...],
                              preferred_element_type=jnp.float32)
        or_ref[...] = jnp.dot(xb, wr_ref[...],
                              preferred_element_type=jnp.float32)
    return pl.pallas_call(
        body,
        grid=(N // MROWS,),
        in_specs=[pl.BlockSpec((MROWS, D), lambda i: (i, 0)),
                  pl.BlockSpec((D, D), lambda i: (0, 0)),
                  pl.BlockSpec((D, D), lambda i: (0, 0))],
        out_specs=[pl.BlockSpec((MROWS, D), lambda i: (i, 0)),
                   pl.BlockSpec((MROWS, D), lambda i: (i, 0))],
        out_shape=[jax.ShapeDtypeStruct((N, D), jnp.float32),
                   jax.ShapeDtypeStruct((N, D), jnp.float32)],
    )(xin, Wl, Wr)


def _tc_merge_matmul(accm, accd, b, Wl, Wr):
    def body(m_ref, d_ref, b_ref, wl_ref, wr_ref, ol_ref, or_ref):
        den = d_ref[0, :, 0:1] + d_ref[1, :, 0:1] + 1e-16
        h = (m_ref[0] + m_ref[1]) / den + b_ref[...]
        h = jnp.maximum(h, 0.1 * h)
        ol_ref[...] = jnp.dot(h, wl_ref[...],
                              preferred_element_type=jnp.float32)
        or_ref[...] = jnp.dot(h, wr_ref[...],
                              preferred_element_type=jnp.float32)
    return pl.pallas_call(
        body,
        grid=(N // MROWS,),
        in_specs=[pl.BlockSpec((NC, MROWS, D), lambda i: (0, i, 0)),
                  pl.BlockSpec((NC, MROWS, 8), lambda i: (0, i, 0)),
                  pl.BlockSpec((1, D), lambda i: (0, 0)),
                  pl.BlockSpec((D, D), lambda i: (0, 0)),
                  pl.BlockSpec((D, D), lambda i: (0, 0))],
        out_specs=[pl.BlockSpec((MROWS, D), lambda i: (i, 0)),
                   pl.BlockSpec((MROWS, D), lambda i: (i, 0))],
        out_shape=[jax.ShapeDtypeStruct((N, D), jnp.float32),
                   jax.ShapeDtypeStruct((N, D), jnp.float32)],
    )(accm, accd, b, Wl, Wr)


def _tc_final(accm, accd, b):
    def body(m_ref, d_ref, b_ref, o_ref):
        den = d_ref[0, :, 0:1] + d_ref[1, :, 0:1] + 1e-16
        z = (m_ref[0] + m_ref[1]) / den + b_ref[...]
        o_ref[...] = 1.0 / (1.0 + jnp.exp(-z))
    return pl.pallas_call(
        body,
        grid=(N // MROWS,),
        in_specs=[pl.BlockSpec((NC, MROWS, D), lambda i: (0, i, 0)),
                  pl.BlockSpec((NC, MROWS, 8), lambda i: (0, i, 0)),
                  pl.BlockSpec((1, D), lambda i: (0, 0))],
        out_specs=pl.BlockSpec((MROWS, D), lambda i: (i, 0)),
        out_shape=jax.ShapeDtypeStruct((N, D), jnp.float32),
    )(accm, accd, b)


def kernel(x, edge_index, Wl1, Wr1, att1, b1, Wl2, Wr2, att2, b2):
    xl1, xr1 = _tc_dual_matmul(x, Wl1, Wr1)
    accm1, accd1 = _sc_edge_phase(xl1, xr1, edge_index, att1)
    xl2, xr2 = _tc_merge_matmul(accm1, accd1, b1.reshape(1, D), Wl2, Wr2)
    accm2, accd2 = _sc_edge_phase(xl2, xr2, edge_index, att2)
    return _tc_final(accm2, accd2, b2.reshape(1, D))
